# Initial kernel scaffold; baseline (speedup 1.0000x reference)
import jax
import jax.numpy as jnp
from jax.experimental import pallas as pl


def kernel(user_hist, target_item, user_table, item_table, W1, b1, W2, b2):
    raise NotImplementedError("write your pallas kernel here")


# speedup vs baseline: 7.2709x; 7.2709x over previous
import jax
import jax.numpy as jnp
from jax.experimental import pallas as pl
from jax.experimental.pallas import tpu as pltpu

VOCAB = 1000
VOCAB_P = 1024
EMBED_DIM = 64
HIDDEN_DIM = 256
SEQ_LENGTH = 50
BATCH = 4096
BLOCK_B = 256


def _din_kernel(hist_ref, tgt_ref, ut_ref, it_ref, w1_ref, b1_ref, w2_ref,
                b2_ref, out_ref):
    hist = hist_ref[...]
    iota = jax.lax.broadcasted_iota(jnp.int32, (1, VOCAB_P), 1)

    counts = jnp.zeros((BLOCK_B, VOCAB_P), jnp.float32)
    for l in range(SEQ_LENGTH):
        col = hist[:, l:l + 1]
        counts = counts + (col == iota).astype(jnp.float32)

    interest = jax.lax.dot(counts, ut_ref[...],
                           precision=jax.lax.Precision.HIGHEST) * (1.0 / SEQ_LENGTH)

    onehot_t = (tgt_ref[...] == iota).astype(jnp.float32)
    temb = jax.lax.dot(onehot_t, it_ref[...],
                       precision=jax.lax.Precision.HIGHEST)

    x = jnp.concatenate([interest, temb], axis=1)
    h = jnp.maximum(
        jax.lax.dot(x, w1_ref[...], precision=jax.lax.Precision.HIGHEST)
        + b1_ref[...], 0.0)
    out = jax.lax.dot(h, w2_ref[...],
                      precision=jax.lax.Precision.HIGHEST) + b2_ref[...]
    out_ref[...] = jax.nn.sigmoid(out)


def kernel(user_hist, target_item, user_table, item_table, W1, b1, W2, b2):
    pad = VOCAB_P - VOCAB
    ut = jnp.pad(user_table, ((0, pad), (0, 0)))
    it = jnp.pad(item_table, ((0, pad), (0, 0)))
    tgt = target_item.reshape(BATCH, 1).astype(jnp.int32)
    hist = user_hist.astype(jnp.int32)

    grid = (BATCH // BLOCK_B,)
    out = pl.pallas_call(
        _din_kernel,
        grid=grid,
        in_specs=[
            pl.BlockSpec((BLOCK_B, SEQ_LENGTH), lambda i: (i, 0)),
            pl.BlockSpec((BLOCK_B, 1), lambda i: (i, 0)),
            pl.BlockSpec((VOCAB_P, EMBED_DIM), lambda i: (0, 0)),
            pl.BlockSpec((VOCAB_P, EMBED_DIM), lambda i: (0, 0)),
            pl.BlockSpec((2 * EMBED_DIM, HIDDEN_DIM), lambda i: (0, 0)),
            pl.BlockSpec((1, HIDDEN_DIM), lambda i: (0, 0)),
            pl.BlockSpec((HIDDEN_DIM, 1), lambda i: (0, 0)),
            pl.BlockSpec((1, 1), lambda i: (0, 0)),
        ],
        out_specs=pl.BlockSpec((BLOCK_B, 1), lambda i: (i, 0)),
        out_shape=jax.ShapeDtypeStruct((BATCH, 1), jnp.float32),
    )(hist, tgt, ut, it, W1, b1.reshape(1, HIDDEN_DIM), W2,
      b2.reshape(1, 1))
    return out


# speedup vs baseline: 8.1874x; 1.1260x over previous
import functools
import jax
import jax.numpy as jnp
import numpy as np
from jax import lax
from jax.experimental import pallas as pl
from jax.experimental.pallas import tpu as pltpu
from jax.experimental.pallas import tpu_sc as plsc

VOCAB = 1000
EMBED_DIM = 64
HIDDEN_DIM = 256
SEQ_LENGTH = 50
BATCH = 4096

NUM_CORES = 2
NUM_SUBCORES = 16
NUM_WORKERS = NUM_CORES * NUM_SUBCORES
BPW = BATCH // NUM_WORKERS
IPW = BPW * SEQ_LENGTH
CHUNK = 128
NCHUNK = IPW // CHUNK

BLOCK_B = 512


def _sc_body(hist_hbm, tgt_hbm, seg_hbm, utab_hbm, itab_hbm,
             psum_hbm, temb_hbm,
             idx_v, seg_v, rows_v, acc_sh, tidx_v, trows_v, sem):
    sid = lax.axis_index("s")
    wid = sid * NUM_CORES + lax.axis_index("c")
    base = wid * BPW
    sbase = sid * BPW

    pltpu.sync_copy(hist_hbm.at[wid], idx_v)
    pltpu.sync_copy(seg_hbm.at[sid], seg_v)

    zero = jnp.zeros((1, 16), jnp.float32)

    @pl.loop(0, BPW)
    def _(r):
        @pl.loop(0, EMBED_DIM, step=16)
        def _(c0):
            rows_v.at[pl.ds(r, 1), pl.ds(c0, 16)][...] = zero

    pltpu.sync_copy(rows_v, acc_sh.at[pl.ds(sbase, BPW)])

    @pl.loop(0, NCHUNK)
    def _(j):
        pltpu.async_copy(utab_hbm.at[idx_v.at[j]], rows_v, sem).wait()
        pltpu.sync_copy(rows_v, acc_sh.at[seg_v.at[j]], add=True)

    pltpu.sync_copy(tgt_hbm.at[wid], tidx_v)
    pltpu.async_copy(itab_hbm.at[tidx_v], trows_v, sem).wait()

    pltpu.sync_copy(acc_sh.at[pl.ds(sbase, BPW)], psum_hbm.at[pl.ds(base, BPW)])
    pltpu.sync_copy(trows_v, temb_hbm.at[pl.ds(base, BPW)])


@jax.jit
def _sc_pool(hist, tgt, seg, utab, itab):
    mesh = plsc.VectorSubcoreMesh(core_axis_name="c", subcore_axis_name="s")
    k = pl.kernel(
        _sc_body,
        out_type=[
            jax.ShapeDtypeStruct((BATCH, EMBED_DIM), jnp.float32),
            jax.ShapeDtypeStruct((BATCH, EMBED_DIM), jnp.float32),
        ],
        mesh=mesh,
        scratch_types=[
            pltpu.VMEM((NCHUNK, CHUNK), jnp.int32),
            pltpu.VMEM((NCHUNK, CHUNK), jnp.int32),
            pltpu.VMEM((CHUNK, EMBED_DIM), jnp.float32),
            pltpu.VMEM_SHARED((NUM_SUBCORES * BPW, EMBED_DIM), jnp.float32),
            pltpu.VMEM((BPW,), jnp.int32),
            pltpu.VMEM((BPW, EMBED_DIM), jnp.float32),
            pltpu.SemaphoreType.DMA,
        ],
        compiler_params=pltpu.CompilerParams(use_tc_tiling_on_sc=False),
    )
    return k(hist, tgt, seg, utab, itab)


def _mlp_kernel(ps_ref, te_ref, w1a_ref, w1b_ref, b1_ref, w2_ref, b2_ref,
                out_ref):
    interest = ps_ref[...] * (1.0 / SEQ_LENGTH)
    h = jnp.maximum(
        jax.lax.dot(interest, w1a_ref[...],
                    precision=jax.lax.Precision.HIGHEST)
        + jax.lax.dot(te_ref[...], w1b_ref[...],
                      precision=jax.lax.Precision.HIGHEST)
        + b1_ref[...], 0.0)
    out = jax.lax.dot(h, w2_ref[...],
                      precision=jax.lax.Precision.HIGHEST) + b2_ref[...]
    out_ref[...] = jax.nn.sigmoid(out)


def _mlp(psum, temb, W1, b1, W2, b2):
    grid = (BATCH // BLOCK_B,)
    return pl.pallas_call(
        _mlp_kernel,
        grid=grid,
        in_specs=[
            pl.BlockSpec((BLOCK_B, EMBED_DIM), lambda i: (i, 0)),
            pl.BlockSpec((BLOCK_B, EMBED_DIM), lambda i: (i, 0)),
            pl.BlockSpec((EMBED_DIM, HIDDEN_DIM), lambda i: (0, 0)),
            pl.BlockSpec((EMBED_DIM, HIDDEN_DIM), lambda i: (0, 0)),
            pl.BlockSpec((1, HIDDEN_DIM), lambda i: (0, 0)),
            pl.BlockSpec((HIDDEN_DIM, 1), lambda i: (0, 0)),
            pl.BlockSpec((1, 1), lambda i: (0, 0)),
        ],
        out_specs=pl.BlockSpec((BLOCK_B, 1), lambda i: (i, 0)),
        out_shape=jax.ShapeDtypeStruct((BATCH, 1), jnp.float32),
    )(psum, temb, W1[:EMBED_DIM], W1[EMBED_DIM:],
      b1.reshape(1, HIDDEN_DIM), W2, b2.reshape(1, 1))


_SEG = np.asarray(
    (np.arange(IPW, dtype=np.int32) // SEQ_LENGTH).reshape(1, NCHUNK, CHUNK)
    + (np.arange(NUM_SUBCORES, dtype=np.int32) * BPW).reshape(
        NUM_SUBCORES, 1, 1))


def kernel(user_hist, target_item, user_table, item_table, W1, b1, W2, b2):
    hist = user_hist.astype(jnp.int32).reshape(NUM_WORKERS, NCHUNK, CHUNK)
    tgt = target_item.astype(jnp.int32).reshape(NUM_WORKERS, BPW)
    seg = jnp.asarray(_SEG)
    psum, temb = _sc_pool(hist, tgt, seg, user_table, item_table)
    return _mlp(psum, temb, W1, b1, W2, b2)


# speedup vs baseline: 9.8116x; 1.1984x over previous
import functools
import jax
import jax.numpy as jnp
import numpy as np
from jax import lax
from jax.experimental import pallas as pl
from jax.experimental.pallas import tpu as pltpu
from jax.experimental.pallas import tpu_sc as plsc

VOCAB = 1000
EMBED_DIM = 64
HIDDEN_DIM = 256
SEQ_LENGTH = 50
BATCH = 4096

NUM_CORES = 2
NUM_SUBCORES = 16
NUM_WORKERS = NUM_CORES * NUM_SUBCORES
BPW = BATCH // NUM_WORKERS
IPW = BPW * SEQ_LENGTH
CHUNK = 128
NCHUNK = IPW // CHUNK

BLOCK_B = 512


def _sc_body(hist_hbm, tgt_hbm, seg_hbm, utab_hbm, itab_hbm,
             psum_hbm, temb_hbm,
             idx_v, seg_v, rows0_v, rows1_v, acc_sh, tidx_v, trows_v,
             sem0, sem1):
    sid = lax.axis_index("s")
    wid = sid * NUM_CORES + lax.axis_index("c")
    base = wid * BPW
    sbase = sid * BPW

    pltpu.sync_copy(hist_hbm.at[wid], idx_v)
    pltpu.sync_copy(seg_hbm.at[sid], seg_v)

    zero = jnp.zeros((1, 16), jnp.float32)

    @pl.loop(0, BPW)
    def _(r):
        @pl.loop(0, EMBED_DIM, step=16)
        def _(c0):
            rows0_v.at[pl.ds(r, 1), pl.ds(c0, 16)][...] = zero

    pltpu.sync_copy(rows0_v, acc_sh.at[pl.ds(sbase, BPW)])

    def gstart(j, buf, sem):
        pltpu.async_copy(utab_hbm.at[idx_v.at[j]], buf, sem)

    def gwait(j, buf, sem):
        pltpu.make_async_copy(utab_hbm.at[idx_v.at[j]], buf, sem).wait()

    gstart(0, rows0_v, sem0)

    @pl.loop(0, NCHUNK, step=2)
    def _(j):
        gstart(j + 1, rows1_v, sem1)
        gwait(j, rows0_v, sem0)
        pltpu.sync_copy(rows0_v, acc_sh.at[seg_v.at[j]], add=True)

        @pl.when(j + 2 < NCHUNK)
        def _():
            gstart(j + 2, rows0_v, sem0)

        gwait(j + 1, rows1_v, sem1)
        pltpu.sync_copy(rows1_v, acc_sh.at[seg_v.at[j + 1]], add=True)

    pltpu.sync_copy(tgt_hbm.at[wid], tidx_v)
    pltpu.async_copy(itab_hbm.at[tidx_v], trows_v, sem0).wait()

    pltpu.sync_copy(acc_sh.at[pl.ds(sbase, BPW)], psum_hbm.at[pl.ds(base, BPW)])
    pltpu.sync_copy(trows_v, temb_hbm.at[pl.ds(base, BPW)])


@jax.jit
def _sc_pool(hist, tgt, seg, utab, itab):
    mesh = plsc.VectorSubcoreMesh(core_axis_name="c", subcore_axis_name="s")
    k = pl.kernel(
        _sc_body,
        out_type=[
            jax.ShapeDtypeStruct((BATCH, EMBED_DIM), jnp.float32),
            jax.ShapeDtypeStruct((BATCH, EMBED_DIM), jnp.float32),
        ],
        mesh=mesh,
        scratch_types=[
            pltpu.VMEM((NCHUNK, CHUNK), jnp.int32),
            pltpu.VMEM((NCHUNK, CHUNK), jnp.int32),
            pltpu.VMEM((CHUNK, EMBED_DIM), jnp.float32),
            pltpu.VMEM((CHUNK, EMBED_DIM), jnp.float32),
            pltpu.VMEM_SHARED((NUM_SUBCORES * BPW, EMBED_DIM), jnp.float32),
            pltpu.VMEM((BPW,), jnp.int32),
            pltpu.VMEM((BPW, EMBED_DIM), jnp.float32),
            pltpu.SemaphoreType.DMA,
            pltpu.SemaphoreType.DMA,
        ],
        compiler_params=pltpu.CompilerParams(use_tc_tiling_on_sc=False),
    )
    return k(hist, tgt, seg, utab, itab)


def _mlp_kernel(ps_ref, te_ref, w1a_ref, w1b_ref, b1_ref, w2_ref, b2_ref,
                out_ref):
    interest = ps_ref[...] * (1.0 / SEQ_LENGTH)
    h = jnp.maximum(
        jax.lax.dot(interest, w1a_ref[...],
                    precision=jax.lax.Precision.HIGHEST)
        + jax.lax.dot(te_ref[...], w1b_ref[...],
                      precision=jax.lax.Precision.HIGHEST)
        + b1_ref[...], 0.0)
    out = jax.lax.dot(h, w2_ref[...],
                      precision=jax.lax.Precision.HIGHEST) + b2_ref[...]
    out_ref[...] = jax.nn.sigmoid(out)


def _mlp(psum, temb, W1, b1, W2, b2):
    grid = (BATCH // BLOCK_B,)
    return pl.pallas_call(
        _mlp_kernel,
        grid=grid,
        in_specs=[
            pl.BlockSpec((BLOCK_B, EMBED_DIM), lambda i: (i, 0)),
            pl.BlockSpec((BLOCK_B, EMBED_DIM), lambda i: (i, 0)),
            pl.BlockSpec((EMBED_DIM, HIDDEN_DIM), lambda i: (0, 0)),
            pl.BlockSpec((EMBED_DIM, HIDDEN_DIM), lambda i: (0, 0)),
            pl.BlockSpec((1, HIDDEN_DIM), lambda i: (0, 0)),
            pl.BlockSpec((HIDDEN_DIM, 1), lambda i: (0, 0)),
            pl.BlockSpec((1, 1), lambda i: (0, 0)),
        ],
        out_specs=pl.BlockSpec((BLOCK_B, 1), lambda i: (i, 0)),
        out_shape=jax.ShapeDtypeStruct((BATCH, 1), jnp.float32),
    )(psum, temb, W1[:EMBED_DIM], W1[EMBED_DIM:],
      b1.reshape(1, HIDDEN_DIM), W2, b2.reshape(1, 1))


_SEG = np.asarray(
    (np.arange(IPW, dtype=np.int32) // SEQ_LENGTH).reshape(1, NCHUNK, CHUNK)
    + (np.arange(NUM_SUBCORES, dtype=np.int32) * BPW).reshape(
        NUM_SUBCORES, 1, 1))


def kernel(user_hist, target_item, user_table, item_table, W1, b1, W2, b2):
    hist = user_hist.astype(jnp.int32).reshape(NUM_WORKERS, NCHUNK, CHUNK)
    tgt = target_item.astype(jnp.int32).reshape(NUM_WORKERS, BPW)
    seg = jnp.asarray(_SEG)
    psum, temb = _sc_pool(hist, tgt, seg, user_table, item_table)
    return _mlp(psum, temb, W1, b1, W2, b2)
